# Initial kernel scaffold; baseline (speedup 1.0000x reference)
import functools

import jax
import jax.numpy as jnp
from jax import lax
from jax.experimental import pallas as pl
from jax.experimental.pallas import tpu as pltpu
from jax.experimental.pallas import tpu_sc as plsc

NC = 2
NS = 16
NW = NC * NS
CHUNK = 128
ROW_BLOCK = 1000


def _mm_bias_body(x_ref, w_ref, b_ref, o_ref):
    o_ref[...] = (
        jnp.dot(x_ref[...], w_ref[...], preferred_element_type=jnp.float32)
        + b_ref[...]
    )


def _mm_bias(x, W, b):
    n, d_in = x.shape
    d_out = W.shape[1]
    grid = n // ROW_BLOCK
    return pl.pallas_call(
        _mm_bias_body,
        grid=(grid,),
        in_specs=[
            pl.BlockSpec((ROW_BLOCK, d_in), lambda i: (i, 0)),
            pl.BlockSpec((d_in, d_out), lambda i: (0, 0)),
            pl.BlockSpec((1, d_out), lambda i: (0, 0)),
        ],
        out_specs=pl.BlockSpec((ROW_BLOCK, d_out), lambda i: (i, 0)),
        out_shape=jax.ShapeDtypeStruct((n, d_out), jnp.float32),
    )(x, W, b.reshape(1, d_out))


def _combine_relu_mm_body(acc_ref, w_ref, b_ref, o_ref):
    x1 = jnp.maximum(acc_ref[0] + acc_ref[1], 0.0)
    o_ref[...] = (
        jnp.dot(x1, w_ref[...], preferred_element_type=jnp.float32) + b_ref[...]
    )


def _combine_relu_mm(parts, W, b):
    _, n, d_in = parts.shape
    d_out = W.shape[1]
    grid = n // ROW_BLOCK
    return pl.pallas_call(
        _combine_relu_mm_body,
        grid=(grid,),
        in_specs=[
            pl.BlockSpec((2, ROW_BLOCK, d_in), lambda i: (0, i, 0)),
            pl.BlockSpec((d_in, d_out), lambda i: (0, 0)),
            pl.BlockSpec((1, d_out), lambda i: (0, 0)),
        ],
        out_specs=pl.BlockSpec((ROW_BLOCK, d_out), lambda i: (i, 0)),
        out_shape=jax.ShapeDtypeStruct((n, d_out), jnp.float32),
    )(parts, W, b.reshape(1, d_out))


def _combine_body(acc_ref, o_ref):
    o_ref[...] = acc_ref[0] + acc_ref[1]


def _combine(parts):
    _, n, d = parts.shape
    grid = n // ROW_BLOCK
    return pl.pallas_call(
        _combine_body,
        grid=(grid,),
        in_specs=[pl.BlockSpec((2, ROW_BLOCK, d), lambda i: (0, i, 0))],
        out_specs=pl.BlockSpec((ROW_BLOCK, d), lambda i: (i, 0)),
        out_shape=jax.ShapeDtypeStruct((n, d), jnp.float32),
    )(parts)


def _make_sc_scatter(n, d, ch_per_tile):
    acc_rows = ((n + 1 + NS - 1) // NS) * NS
    zero_per_tile = acc_rows // NS
    out_per_tile = n // NS
    mesh = plsc.VectorSubcoreMesh(core_axis_name="c", subcore_axis_name="s")

    @functools.partial(
        pl.kernel,
        out_type=jax.ShapeDtypeStruct((NC, n, d), jnp.float32),
        mesh=mesh,
        scratch_types=[
            pltpu.VMEM((CHUNK,), jnp.int32),
            pltpu.VMEM((CHUNK,), jnp.int32),
            pltpu.VMEM((CHUNK, d), jnp.float32),
            pltpu.VMEM_SHARED((acc_rows, d), jnp.float32),
            pltpu.SemaphoreType.DMA,
        ],
    )
    def sc_scatter(support_hbm, src_hbm, dst_hbm, out_hbm,
                   src_v, dst_v, rows_v, acc_sh, sem):
        c = lax.axis_index("c")
        s = lax.axis_index("s")
        t = c * NS + s

        def zbody(i, _):
            r = i // (d // 16)
            col = (i % (d // 16)) * 16
            rows_v[r, pl.ds(col, 16)] = jnp.zeros((16,), jnp.float32)
            return ()
        lax.fori_loop(0, CHUNK * (d // 16), zbody, ())
        for k in range(zero_per_tile // CHUNK):
            pltpu.sync_copy(
                rows_v, acc_sh.at[pl.ds(s * zero_per_tile + k * CHUNK, CHUNK)])
        rem = zero_per_tile % CHUNK
        if rem:
            pltpu.sync_copy(
                rows_v.at[pl.ds(0, rem)],
                acc_sh.at[pl.ds(
                    s * zero_per_tile + (zero_per_tile // CHUNK) * CHUNK, rem)])
        plsc.subcore_barrier()

        def body(j, _):
            pltpu.sync_copy(src_hbm.at[t, j], src_v)
            pltpu.sync_copy(dst_hbm.at[t, j], dst_v)
            pltpu.async_copy(support_hbm.at[src_v], rows_v, sem).wait()
            pltpu.sync_copy(rows_v, acc_sh.at[dst_v], add=True)
            return ()
        lax.fori_loop(0, ch_per_tile, body, ())
        plsc.subcore_barrier()

        n_out_chunks = out_per_tile // CHUNK + (1 if out_per_tile % CHUNK else 0)
        for k in range(n_out_chunks):
            r0 = k * CHUNK
            nr = min(CHUNK, out_per_tile - r0)
            pltpu.sync_copy(
                acc_sh.at[pl.ds(s * out_per_tile + r0, nr)],
                rows_v.at[pl.ds(0, nr)])
            pltpu.sync_copy(
                rows_v.at[pl.ds(0, nr)],
                out_hbm.at[c].at[pl.ds(s * out_per_tile + r0, nr)])

    return sc_scatter


def kernel(x, adj, W1, b1, W2, b2):
    n, d = x.shape
    e = adj.shape[1]
    ch_per_tile = -(-e // (NW * CHUNK))
    e_pad = NW * ch_per_tile * CHUNK

    src = adj[0].astype(jnp.int32)
    dst = adj[1].astype(jnp.int32)
    pad = e_pad - e
    if pad:
        src = jnp.concatenate([src, jnp.zeros((pad,), jnp.int32)])
        dst = jnp.concatenate([dst, jnp.full((pad,), n, jnp.int32)])
    src_t = src.reshape(NW, ch_per_tile, CHUNK)
    dst_t = dst.reshape(NW, ch_per_tile, CHUNK)

    sc_scatter = _make_sc_scatter(n, d, ch_per_tile)

    support1 = _mm_bias(x, W1, b1)
    parts1 = sc_scatter(support1, src_t, dst_t)
    support2 = _combine_relu_mm(parts1, W2, b2)
    parts2 = sc_scatter(support2, src_t, dst_t)
    return _combine(parts2)


# speedup vs baseline: 4.3201x; 4.3201x over previous
import functools

import jax
import jax.numpy as jnp
from jax import lax
from jax.experimental import pallas as pl
from jax.experimental.pallas import tpu as pltpu
from jax.experimental.pallas import tpu_sc as plsc

NC = 2
NS = 16
NW = NC * NS
CHUNK = 128
ROW_BLOCK = 1000


def _mm_bias_body(x_ref, w_ref, b_ref, o_ref):
    o_ref[...] = (
        jnp.dot(x_ref[...], w_ref[...], preferred_element_type=jnp.float32)
        + b_ref[...]
    )


def _mm_bias(x, W, b):
    n, d_in = x.shape
    d_out = W.shape[1]
    grid = n // ROW_BLOCK
    return pl.pallas_call(
        _mm_bias_body,
        grid=(grid,),
        in_specs=[
            pl.BlockSpec((ROW_BLOCK, d_in), lambda i: (i, 0)),
            pl.BlockSpec((d_in, d_out), lambda i: (0, 0)),
            pl.BlockSpec((1, d_out), lambda i: (0, 0)),
        ],
        out_specs=pl.BlockSpec((ROW_BLOCK, d_out), lambda i: (i, 0)),
        out_shape=jax.ShapeDtypeStruct((n, d_out), jnp.float32),
    )(x, W, b.reshape(1, d_out))


def _combine_relu_mm_body(acc_ref, w_ref, b_ref, o_ref):
    x1 = jnp.maximum(acc_ref[0] + acc_ref[1], 0.0)
    o_ref[...] = (
        jnp.dot(x1, w_ref[...], preferred_element_type=jnp.float32) + b_ref[...]
    )


def _combine_relu_mm(parts, W, b, n):
    d_in = parts.shape[2]
    d_out = W.shape[1]
    grid = n // ROW_BLOCK
    return pl.pallas_call(
        _combine_relu_mm_body,
        grid=(grid,),
        in_specs=[
            pl.BlockSpec((2, ROW_BLOCK, d_in), lambda i: (0, i, 0)),
            pl.BlockSpec((d_in, d_out), lambda i: (0, 0)),
            pl.BlockSpec((1, d_out), lambda i: (0, 0)),
        ],
        out_specs=pl.BlockSpec((ROW_BLOCK, d_out), lambda i: (i, 0)),
        out_shape=jax.ShapeDtypeStruct((n, d_out), jnp.float32),
    )(parts, W, b.reshape(1, d_out))


def _combine_body(acc_ref, o_ref):
    o_ref[...] = acc_ref[0] + acc_ref[1]


def _combine(parts, n):
    d = parts.shape[2]
    grid = n // ROW_BLOCK
    return pl.pallas_call(
        _combine_body,
        grid=(grid,),
        in_specs=[pl.BlockSpec((2, ROW_BLOCK, d), lambda i: (0, i, 0))],
        out_specs=pl.BlockSpec((ROW_BLOCK, d), lambda i: (i, 0)),
        out_shape=jax.ShapeDtypeStruct((n, d), jnp.float32),
    )(parts)


def _acc_rows(n):
    return ((n + 1 + NS * CHUNK - 1) // (NS * CHUNK)) * (NS * CHUNK)


def _make_sc_scatter(n, d, ch_per_tile):
    acc_rows = _acc_rows(n)
    per_tile = acc_rows // NS
    mesh = plsc.VectorSubcoreMesh(core_axis_name="c", subcore_axis_name="s")

    @functools.partial(
        pl.kernel,
        out_type=jax.ShapeDtypeStruct((NC, acc_rows, d), jnp.float32),
        mesh=mesh,
        scratch_types=[
            pltpu.VMEM((ch_per_tile, CHUNK), jnp.int32),
            pltpu.VMEM((ch_per_tile, CHUNK), jnp.int32),
            pltpu.VMEM((CHUNK, d), jnp.float32),
            pltpu.VMEM_SHARED((acc_rows, d), jnp.float32),
            pltpu.SemaphoreType.DMA,
        ],
    )
    def sc_scatter(support_hbm, src_hbm, dst_hbm, out_hbm,
                   src_v, dst_v, rows_v, acc_sh, sem):
        c = lax.axis_index("c")
        s = lax.axis_index("s")
        t = c * NS + s

        pltpu.sync_copy(src_hbm.at[t], src_v)
        pltpu.sync_copy(dst_hbm.at[t], dst_v)

        def zbody(i, _):
            r = i // (d // 16)
            col = (i % (d // 16)) * 16
            rows_v[r, pl.ds(col, 16)] = jnp.zeros((16,), jnp.float32)
            return ()
        lax.fori_loop(0, CHUNK * (d // 16), zbody, ())
        for k in range(per_tile // CHUNK):
            pltpu.sync_copy(
                rows_v, acc_sh.at[pl.ds(s * per_tile + k * CHUNK, CHUNK)])
        plsc.subcore_barrier()

        def body(j, _):
            pltpu.async_copy(support_hbm.at[src_v.at[j]], rows_v, sem).wait()
            pltpu.sync_copy(rows_v, acc_sh.at[dst_v.at[j]], add=True)
            return ()
        lax.fori_loop(0, ch_per_tile, body, ())
        plsc.subcore_barrier()

        for k in range(per_tile // CHUNK):
            r0 = s * per_tile + k * CHUNK
            pltpu.sync_copy(acc_sh.at[pl.ds(r0, CHUNK)], rows_v)
            pltpu.sync_copy(rows_v, out_hbm.at[c].at[pl.ds(r0, CHUNK)])

    return sc_scatter


def kernel(x, adj, W1, b1, W2, b2):
    n, d = x.shape
    e = adj.shape[1]
    ch_per_tile = -(-e // (NW * CHUNK))
    e_pad = NW * ch_per_tile * CHUNK

    src = adj[0].astype(jnp.int32)
    dst = adj[1].astype(jnp.int32)
    pad = e_pad - e
    if pad:
        src = jnp.concatenate([src, jnp.zeros((pad,), jnp.int32)])
        dst = jnp.concatenate([dst, jnp.full((pad,), n, jnp.int32)])
    src_t = src.reshape(NW, ch_per_tile, CHUNK)
    dst_t = dst.reshape(NW, ch_per_tile, CHUNK)

    sc_scatter = _make_sc_scatter(n, d, ch_per_tile)

    support1 = _mm_bias(x, W1, b1)
    parts1 = sc_scatter(support1, src_t, dst_t)
    support2 = _combine_relu_mm(parts1, W2, b2, n)
    parts2 = sc_scatter(support2, src_t, dst_t)
    return _combine(parts2, n)
